# baseline (speedup 1.0000x reference)
import functools

import jax
import jax.numpy as jnp
from jax import lax
from jax.experimental import pallas as pl
from jax.experimental.pallas import tpu as pltpu
from jax.experimental.pallas import tpu_sc as plsc

NUM_LEVELS = 100000
TPAD = NUM_LEVELS + 16

NC = 2
NS = 16
L = 16
NW = NC * NS
BC = 256
IW = 128


@functools.lru_cache(maxsize=None)
def _build(C: int, R: int):
    assert R % (NW * BC) == 0 and (C * BC) % IW == 0
    cols_w = R // NW
    n_blk = cols_w // BC
    n_chunks = C * (BC // L)
    n_rows = (C * BC) // IW
    cpr = IW // L

    mesh = plsc.VectorSubcoreMesh(core_axis_name="c", subcore_axis_name="s")

    @functools.partial(
        pl.kernel,
        mesh=mesh,
        compiler_params=pltpu.CompilerParams(needs_layout_passes=False),
        out_type=jax.ShapeDtypeStruct((C, R), jnp.float32),
        scratch_types=[
            pltpu.VMEM_SHARED((NUM_LEVELS,), jnp.float32),
            pltpu.VMEM((n_rows, IW), jnp.int32),
            pltpu.VMEM((n_rows, IW), jnp.float32),
            pltpu.VMEM((C, BC), jnp.int32),
            pltpu.VMEM((C, BC), jnp.int32),
            pltpu.VMEM((C, BC), jnp.float32),
            pltpu.VMEM((L,), jnp.float32),
            pltpu.SemaphoreType.DMA,
            pltpu.SemaphoreType.DMA,
            pltpu.SemaphoreType.DMA,
        ],
    )
    def sc_gather(fv_hbm, table_hbm, int_hbm, out_hbm,
                  table_sh, flat_idx, flat_val, idx_a, idx_b, out_v, int_v,
                  sem_t, sem_i, sem_g):
        cid = lax.axis_index("c")
        sid = lax.axis_index("s")
        wid = sid * NC + cid
        base = wid * cols_w
        idx_bufs = (idx_a, idx_b)
        cps = [
            pltpu.async_copy(
                fv_hbm.at[:, pl.ds(base + b * BC, BC)], idx_bufs[b], sem_i)
            for b in range(n_blk)
        ]
        cp_s = pltpu.async_copy(int_hbm, int_v, sem_t)

        @pl.when(sid == 0)
        def _():
            pltpu.sync_copy(table_hbm, table_sh)

        plsc.subcore_barrier()
        cp_s.wait()
        inter = int_v[...]

        for b in range(n_blk):
            cps[b].wait()
            idx_v = idx_bufs[b]

            @plsc.parallel_loop(0, n_chunks, unroll=8)
            def repack(i):
                r = i // (BC // L)
                c = i % (BC // L)
                idx = idx_v[r, pl.ds(c * L, L)]
                valid = (idx >= 0) & (idx < NUM_LEVELS)
                idx2 = jnp.where(valid, idx, 0)
                flat_idx[i // cpr, pl.ds((i % cpr) * L, L)] = idx2

            gcps = [
                pltpu.async_copy(
                    table_sh.at[flat_idx.at[j]], flat_val.at[j], sem_g)
                for j in range(n_rows)
            ]
            for cp in gcps:
                cp.wait()

            @plsc.parallel_loop(0, n_chunks, unroll=8)
            def emit(i):
                vals = flat_val[i // cpr, pl.ds((i % cpr) * L, L)]
                r = i // (BC // L)
                c = i % (BC // L)
                idx = idx_v[r, pl.ds(c * L, L)]
                valid = (idx >= 0) & (idx < NUM_LEVELS)
                vals = jnp.where(valid, vals, jnp.zeros((L,), jnp.float32))
                out_v[r, pl.ds(c * L, L)] = vals + inter

            pltpu.sync_copy(out_v, out_hbm.at[:, pl.ds(base + b * BC, BC)])

    return sc_gather


def kernel(feature_vals, re_loc, intercept):
    R, C = feature_vals.shape
    fvT = feature_vals.astype(jnp.int32).T
    ivec = jnp.full((L,), intercept, jnp.float32)
    outT = _build(C, R)(fvT, re_loc, ivec)
    return outT.T
